# Initial kernel scaffold; baseline (speedup 1.0000x reference)
import jax
import jax.numpy as jnp
from jax.experimental import pallas as pl


def kernel(flat_values, cu_seqlens):
    raise NotImplementedError("write your pallas kernel here")


# speedup vs baseline: 1977.9819x; 1977.9819x over previous
import functools

import jax
import jax.numpy as jnp
from jax import lax
from jax.experimental import pallas as pl
from jax.experimental.pallas import tpu as pltpu
from jax.experimental.pallas import tpu_sc as plsc

TOTAL_TOKENS = 2097152
BATCH = 1024
MAX_LEN = 4096

NUM_CORES = 2
NUM_SUBCORES = 16
LANES = 16
NW = NUM_CORES * NUM_SUBCORES
ROWS_PER_W = BATCH // NW

WIN = MAX_LEN + LANES
BUF = WIN + MAX_LEN
CU_BUF = BATCH + LANES


def _ts_kernel(flat_hbm, cu_hbm, out_hbm, cu_v, win_v, row_v,
               cu_sem, win_sem, out_sem):
    wid = lax.axis_index("s") * NUM_CORES + lax.axis_index("c")
    row0 = wid * ROWS_PER_W
    iota = lax.iota(jnp.int32, LANES)

    pltpu.async_copy(cu_hbm, cu_v.at[pl.ds(0, BATCH + 1)], cu_sem).wait()

    def row_body(r, carry):
        b = row0 + r
        se = cu_v[pl.ds(b, LANES)]
        start = se[0]
        end = se[1]
        seq_len = jnp.minimum(end - start, MAX_LEN)
        base = jnp.minimum((start >> 3) << 3, TOTAL_TOKENS - WIN)
        base = pl.multiple_of(base, 8)
        off = start - base

        pltpu.async_copy(flat_hbm.at[pl.ds(base, WIN)], win_v.at[pl.ds(0, WIN)],
                         win_sem).wait()

        def vec_body(i, c):
            v = win_v[pl.ds(off + i * LANES, LANES)]
            ok = (iota < seq_len - i * LANES) & (v == v)
            row_v[pl.ds(i * LANES, LANES)] = jnp.where(ok, v, jnp.float32(0.0))
            return c

        lax.fori_loop(0, MAX_LEN // LANES, vec_body, 0)
        pltpu.async_copy(row_v, out_hbm.at[b], out_sem).wait()
        return carry

    lax.fori_loop(0, ROWS_PER_W, row_body, 0)


@functools.partial(jax.jit, static_argnames=())
def kernel(flat_values, cu_seqlens):
    mesh = plsc.VectorSubcoreMesh(core_axis_name="c", subcore_axis_name="s")
    run = pl.kernel(
        _ts_kernel,
        out_type=jax.ShapeDtypeStruct((BATCH, MAX_LEN), jnp.float32),
        mesh=mesh,
        scratch_types=[
            pltpu.VMEM((CU_BUF,), jnp.int32),
            pltpu.VMEM((BUF,), jnp.float32),
            pltpu.VMEM((MAX_LEN,), jnp.float32),
            pltpu.SemaphoreType.DMA,
            pltpu.SemaphoreType.DMA,
            pltpu.SemaphoreType.DMA,
        ],
    )
    return run(flat_values, cu_seqlens)


# speedup vs baseline: 3599.6351x; 1.8199x over previous
import functools

import jax
import jax.numpy as jnp
from jax import lax
from jax.experimental import pallas as pl
from jax.experimental.pallas import tpu as pltpu
from jax.experimental.pallas import tpu_sc as plsc

TOTAL_TOKENS = 2097152
BATCH = 1024
MAX_LEN = 4096

NUM_CORES = 2
NUM_SUBCORES = 16
LANES = 16
NW = NUM_CORES * NUM_SUBCORES
ROWS_PER_W = BATCH // NW
PAIRS = ROWS_PER_W // 2

UNROLL = 8
GROUP = UNROLL * LANES
NGROUPS = MAX_LEN // GROUP

WIN = MAX_LEN + LANES
BUF = WIN + MAX_LEN + GROUP
CU_BUF = BATCH + LANES


def _row_params(cu_v, b):
    se = cu_v[pl.ds(b, LANES)]
    start = se[0]
    end = se[1]
    seq_len = jnp.minimum(end - start, MAX_LEN)
    base = jnp.minimum((start >> 3) << 3, TOTAL_TOKENS - WIN)
    base = pl.multiple_of(base, 8)
    return start, seq_len, base


def _issue_win(flat_hbm, cu_v, win_ref, sem, b):
    _, _, base = _row_params(cu_v, b)
    pltpu.async_copy(flat_hbm.at[pl.ds(base, WIN)], win_ref.at[pl.ds(0, WIN)], sem)


def _compute_row(cu_v, win_ref, row_ref, b, iota, zeros):
    start, seq_len, base = _row_params(cu_v, b)
    off = start - base
    ngrp = (seq_len + (GROUP - 1)) // GROUP

    def copy_body(g, c):
        e0 = g * GROUP
        for j in range(UNROLL):
            e = e0 + j * LANES
            v = win_ref[pl.ds(off + e, LANES)]
            ok = (iota < seq_len - e) & (v == v)
            row_ref[pl.ds(e, LANES)] = jnp.where(ok, v, jnp.float32(0.0))
        return c

    def zero_body(g, c):
        e0 = g * GROUP
        for j in range(UNROLL):
            row_ref[pl.ds(e0 + j * LANES, LANES)] = zeros
        return c

    lax.fori_loop(0, ngrp, copy_body, 0)
    lax.fori_loop(ngrp, NGROUPS, zero_body, 0)


def _ts_kernel(flat_hbm, cu_hbm, out_hbm, cu_v, win0, win1, row0, row1,
               cu_sem, win_sem, out_sem):
    wid = lax.axis_index("s") * NUM_CORES + lax.axis_index("c")
    row_base = wid * ROWS_PER_W
    iota = lax.iota(jnp.int32, LANES)
    zeros = jnp.zeros((LANES,), jnp.float32)

    pltpu.async_copy(cu_hbm, cu_v.at[pl.ds(0, BATCH + 1)], cu_sem).wait()

    def wait_win():
        pltpu.make_async_copy(flat_hbm.at[pl.ds(0, WIN)],
                              win0.at[pl.ds(0, WIN)], win_sem).wait()

    def wait_out():
        pltpu.make_async_copy(row0, out_hbm.at[0], out_sem).wait()

    _issue_win(flat_hbm, cu_v, win0, win_sem, row_base)

    def pair_body(k, carry):
        ra = row_base + 2 * k
        rb = ra + 1
        _issue_win(flat_hbm, cu_v, win1, win_sem, rb)

        @pl.when(k >= 1)
        def _():
            wait_out()
            wait_out()

        wait_win()
        _compute_row(cu_v, win0, row0, ra, iota, zeros)
        pltpu.async_copy(row0, out_hbm.at[ra], out_sem)

        @pl.when(k < PAIRS - 1)
        def _():
            _issue_win(flat_hbm, cu_v, win0, win_sem, rb + 1)

        wait_win()
        _compute_row(cu_v, win1, row1, rb, iota, zeros)
        pltpu.async_copy(row1, out_hbm.at[rb], out_sem)
        return carry

    lax.fori_loop(0, PAIRS, pair_body, 0)
    wait_out()
    wait_out()


@functools.partial(jax.jit, static_argnames=())
def kernel(flat_values, cu_seqlens):
    mesh = plsc.VectorSubcoreMesh(core_axis_name="c", subcore_axis_name="s")
    run = pl.kernel(
        _ts_kernel,
        out_type=jax.ShapeDtypeStruct((BATCH, MAX_LEN), jnp.float32),
        mesh=mesh,
        scratch_types=[
            pltpu.VMEM((CU_BUF,), jnp.int32),
            pltpu.VMEM((BUF,), jnp.float32),
            pltpu.VMEM((BUF,), jnp.float32),
            pltpu.VMEM((MAX_LEN,), jnp.float32),
            pltpu.VMEM((MAX_LEN,), jnp.float32),
            pltpu.SemaphoreType.DMA,
            pltpu.SemaphoreType.DMA,
            pltpu.SemaphoreType.DMA,
        ],
    )
    return run(flat_values, cu_seqlens)


# speedup vs baseline: 3823.4435x; 1.0622x over previous
import functools

import jax
import jax.numpy as jnp
from jax import lax
from jax.experimental import pallas as pl
from jax.experimental.pallas import tpu as pltpu
from jax.experimental.pallas import tpu_sc as plsc

TOTAL_TOKENS = 2097152
BATCH = 1024
MAX_LEN = 4096

NUM_CORES = 2
NUM_SUBCORES = 16
LANES = 16
NW = NUM_CORES * NUM_SUBCORES
ROWS_PER_W = BATCH // NW
PAIRS = ROWS_PER_W // 2

UNROLL = 8
GROUP = UNROLL * LANES
NGROUPS = MAX_LEN // GROUP

WIN = MAX_LEN + LANES
WIN_CLASSES = (512 + LANES, 1024 + LANES, 2048 + LANES, WIN)
BUF = WIN + MAX_LEN + GROUP
CU_BUF = BATCH + LANES


def _row_params(cu_v, b):
    se = cu_v[pl.ds(b, LANES)]
    start = se[0]
    end = se[1]
    seq_len = jnp.minimum(end - start, MAX_LEN)
    base0 = (start >> 3) << 3
    return start, seq_len, base0


def _class_preds(seq_len):
    preds = []
    lo = -1
    for w in WIN_CLASSES:
        hi = w - LANES
        preds.append((seq_len > lo) & (seq_len <= hi) if lo >= 0 else seq_len <= hi)
        lo = hi
    return preds


def _issue_win(flat_hbm, cu_v, win_ref, sem, b):
    _, seq_len, base0 = _row_params(cu_v, b)
    for pred, w in zip(_class_preds(seq_len), WIN_CLASSES):
        @pl.when(pred)
        def _(w=w):
            base = pl.multiple_of(jnp.minimum(base0, TOTAL_TOKENS - w), 8)
            pltpu.async_copy(flat_hbm.at[pl.ds(base, w)],
                             win_ref.at[pl.ds(0, w)], sem)


def _wait_win(flat_hbm, cu_v, win_ref, sem, b):
    _, seq_len, _ = _row_params(cu_v, b)
    for pred, w in zip(_class_preds(seq_len), WIN_CLASSES):
        @pl.when(pred)
        def _(w=w):
            pltpu.make_async_copy(flat_hbm.at[pl.ds(0, w)],
                                  win_ref.at[pl.ds(0, w)], sem).wait()


def _compute_row(cu_v, win_ref, row_ref, b, iota, zeros):
    start, seq_len, base0 = _row_params(cu_v, b)
    base = base0
    for pred, w in zip(_class_preds(seq_len), WIN_CLASSES):
        base = jnp.where(pred, jnp.minimum(base0, TOTAL_TOKENS - w), base)
    off = start - base
    nfull = seq_len // GROUP
    ngrp = (seq_len + (GROUP - 1)) // GROUP

    def full_body(g, c):
        e0 = g * GROUP
        for j in range(UNROLL):
            e = e0 + j * LANES
            v = win_ref[pl.ds(off + e, LANES)]
            row_ref[pl.ds(e, LANES)] = jnp.where(v == v, v, jnp.float32(0.0))
        return c

    def zero_body(g, c):
        e0 = g * GROUP
        for j in range(UNROLL):
            row_ref[pl.ds(e0 + j * LANES, LANES)] = zeros
        return c

    lax.fori_loop(0, nfull, full_body, 0)

    @pl.when(ngrp > nfull)
    def _():
        e0 = nfull * GROUP
        for j in range(UNROLL):
            e = e0 + j * LANES
            v = win_ref[pl.ds(off + e, LANES)]
            ok = (iota < seq_len - e) & (v == v)
            row_ref[pl.ds(e, LANES)] = jnp.where(ok, v, jnp.float32(0.0))

    lax.fori_loop(ngrp, NGROUPS, zero_body, 0)


def _ts_kernel(flat_hbm, cu_hbm, out_hbm, cu_v, win0, win1, row0, row1,
               cu_sem, win_sem0, win_sem1, out_sem0, out_sem1):
    wid = lax.axis_index("s") * NUM_CORES + lax.axis_index("c")
    row_base = wid * ROWS_PER_W
    iota = lax.iota(jnp.int32, LANES)
    zeros = jnp.zeros((LANES,), jnp.float32)

    pltpu.async_copy(cu_hbm, cu_v.at[pl.ds(0, BATCH + 1)], cu_sem).wait()

    def wait_out(row_ref, sem):
        pltpu.make_async_copy(row_ref, out_hbm.at[0], sem).wait()

    _issue_win(flat_hbm, cu_v, win0, win_sem0, row_base)

    def pair_body(k, carry):
        ra = row_base + 2 * k
        rb = ra + 1
        _issue_win(flat_hbm, cu_v, win1, win_sem1, rb)

        @pl.when(k >= 1)
        def _():
            wait_out(row0, out_sem0)
            wait_out(row1, out_sem1)

        _wait_win(flat_hbm, cu_v, win0, win_sem0, ra)
        _compute_row(cu_v, win0, row0, ra, iota, zeros)
        pltpu.async_copy(row0, out_hbm.at[ra], out_sem0)

        @pl.when(k < PAIRS - 1)
        def _():
            _issue_win(flat_hbm, cu_v, win0, win_sem0, rb + 1)

        _wait_win(flat_hbm, cu_v, win1, win_sem1, rb)
        _compute_row(cu_v, win1, row1, rb, iota, zeros)
        pltpu.async_copy(row1, out_hbm.at[rb], out_sem1)
        return carry

    lax.fori_loop(0, PAIRS, pair_body, 0)
    wait_out(row0, out_sem0)
    wait_out(row1, out_sem1)


@functools.partial(jax.jit, static_argnames=())
def kernel(flat_values, cu_seqlens):
    mesh = plsc.VectorSubcoreMesh(core_axis_name="c", subcore_axis_name="s")
    run = pl.kernel(
        _ts_kernel,
        out_type=jax.ShapeDtypeStruct((BATCH, MAX_LEN), jnp.float32),
        mesh=mesh,
        scratch_types=[
            pltpu.VMEM((CU_BUF,), jnp.int32),
            pltpu.VMEM((BUF,), jnp.float32),
            pltpu.VMEM((BUF,), jnp.float32),
            pltpu.VMEM((MAX_LEN,), jnp.float32),
            pltpu.VMEM((MAX_LEN,), jnp.float32),
            pltpu.SemaphoreType.DMA,
            pltpu.SemaphoreType.DMA,
            pltpu.SemaphoreType.DMA,
            pltpu.SemaphoreType.DMA,
            pltpu.SemaphoreType.DMA,
        ],
    )
    return run(flat_values, cu_seqlens)
